# baseline (speedup 1.0000x reference)
import functools

import jax
import jax.numpy as jnp
from jax import lax
from jax.experimental import pallas as pl
from jax.experimental.pallas import tpu as pltpu
from jax.experimental.pallas import tpu_sc as plsc

DIM = 64
B = 1024
L = 200
N = B * L
TILE = 128
NTILES = N // TILE

_info = plsc.get_sparse_core_info()
NC, NS = _info.num_cores, _info.num_subcores
NW = NC * NS
TILES_PER_W = NTILES // NW


@functools.partial(
    pl.kernel,
    out_type=jax.ShapeDtypeStruct((NTILES, TILE, DIM), jnp.float32),
    mesh=plsc.VectorSubcoreMesh(core_axis_name="c", subcore_axis_name="s"),
    compiler_params=pltpu.CompilerParams(use_tc_tiling_on_sc=False),
    scratch_types=[
        pltpu.VMEM((1, TILES_PER_W, TILE), jnp.int32),
        pltpu.VMEM((2, TILE, DIM), jnp.float32),
        pltpu.SemaphoreType.DMA,
        pltpu.SemaphoreType.DMA,
    ],
)
def _gather_kernel(table_hbm, idx_hbm, out_hbm, idx_v, rows_v, gsem, wsem):
    wid = lax.axis_index("s") * NC + lax.axis_index("c")
    tbase = wid * TILES_PER_W
    pltpu.sync_copy(idx_hbm.at[pl.ds(wid, 1)], idx_v)
    idx2d = idx_v.at[0]

    pltpu.async_copy(table_hbm.at[idx2d.at[0]], rows_v.at[0], gsem).wait()

    @pl.loop(0, TILES_PER_W - 1)
    def _(j):
        slot = lax.rem(j, 2)
        nxt = 1 - slot
        gather = pltpu.async_copy(
            table_hbm.at[idx2d.at[j + 1]], rows_v.at[nxt], gsem
        )
        write = pltpu.async_copy(rows_v.at[slot], out_hbm.at[tbase + j], wsem)
        gather.wait()
        write.wait()

    last = TILES_PER_W - 1
    pltpu.sync_copy(rows_v.at[lax.rem(last, 2)], out_hbm.at[tbase + last])


def kernel(mode, word_inputs, word_seq_lengths, char_inputs, char_seq_lengths,
           char_seq_recover, word_embedding_weight):
    idx = word_inputs.astype(jnp.int32).reshape(NW, TILES_PER_W, TILE)
    out = _gather_kernel(word_embedding_weight, idx)
    return out.reshape(B, L, DIM)


# speedup vs baseline: 1.0313x; 1.0313x over previous
import functools

import jax
import jax.numpy as jnp
from jax import lax
from jax.experimental import pallas as pl
from jax.experimental.pallas import tpu as pltpu
from jax.experimental.pallas import tpu_sc as plsc

DIM = 64
B = 1024
L = 200
N = B * L

_info = plsc.get_sparse_core_info()
NC, NS = _info.num_cores, _info.num_subcores
NW = NC * NS
PER_W = N // NW
CHUNK = 800
NCHUNK = PER_W // CHUNK
NBUF = 2


@functools.partial(
    pl.kernel,
    out_type=jax.ShapeDtypeStruct((NW * NCHUNK, CHUNK, DIM), jnp.float32),
    mesh=plsc.VectorSubcoreMesh(core_axis_name="c", subcore_axis_name="s"),
    compiler_params=pltpu.CompilerParams(use_tc_tiling_on_sc=False),
    scratch_types=[
        pltpu.VMEM((PER_W,), jnp.int32),
        pltpu.VMEM((NBUF, CHUNK, DIM), jnp.float32),
        pltpu.SemaphoreType.DMA,
        pltpu.SemaphoreType.DMA,
    ],
)
def _gather_kernel(table_hbm, idx_hbm, out_hbm, idx_v, rows_v, gsem, wsem):
    wid = lax.axis_index("s") * NC + lax.axis_index("c")
    cbase = wid * NCHUNK
    pltpu.sync_copy(idx_hbm.at[wid], idx_v)

    for b in range(NBUF):
        pltpu.async_copy(
            table_hbm.at[idx_v.at[pl.ds(b * CHUNK, CHUNK)]], rows_v.at[b], gsem
        )

    @pl.loop(0, NCHUNK)
    def _(j):
        slot = lax.rem(j, NBUF)
        pltpu.make_async_copy(
            table_hbm.at[idx_v.at[pl.ds(0, CHUNK)]], rows_v.at[slot], gsem
        ).wait()
        write = pltpu.async_copy(rows_v.at[slot], out_hbm.at[cbase + j], wsem)

        @pl.when(j + NBUF < NCHUNK)
        def _():
            write.wait()
            pltpu.async_copy(
                table_hbm.at[idx_v.at[pl.ds((j + NBUF) * CHUNK, CHUNK)]],
                rows_v.at[slot],
                gsem,
            )

    for b in range(NBUF):
        pltpu.make_async_copy(rows_v.at[b], out_hbm.at[cbase], wsem).wait()


def kernel(mode, word_inputs, word_seq_lengths, char_inputs, char_seq_lengths,
           char_seq_recover, word_embedding_weight):
    idx = word_inputs.astype(jnp.int32).reshape(NW, PER_W)
    out = _gather_kernel(word_embedding_weight, idx)
    return out.reshape(B, L, DIM)
